# Initial kernel scaffold; baseline (speedup 1.0000x reference)
import functools

import jax
import jax.numpy as jnp
from jax import lax
from jax.experimental import pallas as pl
from jax.experimental.pallas import tpu as pltpu
from jax.experimental.pallas import tpu_sc as plsc

N = 10000
E = 320000
D = 128

NC = 2
NS = 16
NW = NC * NS
B = 128
CHUNKS_PER_TILE = 79
E_PAD = NW * B * CHUNKS_PER_TILE
NPAD = 10016
ROWS_PER_SUB = NPAD // NS
DEGW = 16


def _seg_sum_call(feat, src_p, dst_p, zacc, zdeg, ones_blk, with_deg):
    mesh = plsc.VectorSubcoreMesh(core_axis_name="c", subcore_axis_name="s")
    outs = [jax.ShapeDtypeStruct((NC, NPAD, D), jnp.float32)]
    scratch = [
        pltpu.VMEM((B,), jnp.int32),
        pltpu.VMEM((B,), jnp.int32),
        pltpu.VMEM((B, D), jnp.float32),
        pltpu.VMEM_SHARED((NPAD, D), jnp.float32),
        pltpu.SemaphoreType.DMA,
    ]
    if with_deg:
        outs.append(jax.ShapeDtypeStruct((NC, NPAD, DEGW), jnp.float32))
        scratch += [
            pltpu.VMEM((B, DEGW), jnp.float32),
            pltpu.VMEM_SHARED((NPAD, DEGW), jnp.float32),
        ]

    def body(*refs):
        if with_deg:
            (feat_h, src_h, dst_h, zacc_h, zdeg_h, ones_h,
             acc_out, deg_out,
             src_v, dst_v, rows_v, acc_sh, sem, ones_v, deg_sh) = refs
        else:
            (feat_h, src_h, dst_h, zacc_h,
             acc_out,
             src_v, dst_v, rows_v, acc_sh, sem) = refs
        cid = lax.axis_index("c")
        sid = lax.axis_index("s")
        wid = sid * NC + cid

        rz = ROWS_PER_SUB
        pltpu.sync_copy(zacc_h.at[pl.ds(sid * rz, rz)],
                        acc_sh.at[pl.ds(sid * rz, rz)])
        if with_deg:
            pltpu.sync_copy(zdeg_h.at[pl.ds(sid * rz, rz)],
                            deg_sh.at[pl.ds(sid * rz, rz)])
            pltpu.sync_copy(ones_h, ones_v)
        plsc.subcore_barrier()

        base = wid * (CHUNKS_PER_TILE * B)

        @pl.loop(0, CHUNKS_PER_TILE)
        def _(j):
            off = base + j * B
            pltpu.sync_copy(src_h.at[pl.ds(off, B)], src_v)
            pltpu.sync_copy(dst_h.at[pl.ds(off, B)], dst_v)
            pltpu.async_copy(feat_h.at[src_v], rows_v, sem).wait()
            pltpu.sync_copy(rows_v, acc_sh.at[dst_v], add=True)
            if with_deg:
                pltpu.sync_copy(ones_v, deg_sh.at[dst_v], add=True)

        plsc.subcore_barrier()

        pltpu.sync_copy(acc_sh.at[pl.ds(sid * rz, rz)],
                        acc_out.at[cid, pl.ds(sid * rz, rz)])
        if with_deg:
            pltpu.sync_copy(deg_sh.at[pl.ds(sid * rz, rz)],
                            deg_out.at[cid, pl.ds(sid * rz, rz)])

    k = pl.kernel(body, out_type=tuple(outs), mesh=mesh,
                  scratch_types=tuple(scratch))
    if with_deg:
        return k(feat, src_p, dst_p, zacc, zdeg, ones_blk)
    return (k(feat, src_p, dst_p, zacc),)


_R = 400


def _tc_layer1(sums, deg, x, W_l, b_l, W_r):
    def body(s_ref, d_ref, x_ref, wl_ref, bl_ref, wr_ref, o_ref):
        s = s_ref[0] + s_ref[1]
        dg = d_ref[0, :, 0:1] + d_ref[1, :, 0:1]
        mean = s / jnp.maximum(dg, 1.0)
        acc = jnp.dot(mean, wl_ref[...], preferred_element_type=jnp.float32)
        acc = acc + jnp.dot(x_ref[...], wr_ref[...],
                            preferred_element_type=jnp.float32)
        o_ref[...] = jnp.maximum(acc + bl_ref[...], 0.0)

    return pl.pallas_call(
        body,
        grid=(N // _R,),
        in_specs=[
            pl.BlockSpec((NC, _R, D), lambda i: (0, i, 0)),
            pl.BlockSpec((NC, _R, DEGW), lambda i: (0, i, 0)),
            pl.BlockSpec((_R, D), lambda i: (i, 0)),
            pl.BlockSpec((D, D), lambda i: (0, 0)),
            pl.BlockSpec((1, D), lambda i: (0, 0)),
            pl.BlockSpec((D, D), lambda i: (0, 0)),
        ],
        out_specs=pl.BlockSpec((_R, D), lambda i: (i, 0)),
        out_shape=jax.ShapeDtypeStruct((N, D), jnp.float32),
    )(sums, deg, x, W_l, b_l.reshape(1, D), W_r)


def _tc_layer2(sums, deg, h, W_l, b_l, W_r, W_lin, b_lin):
    def body(s_ref, d_ref, h_ref, wl_ref, bl_ref, wr_ref, wo_ref, bo_ref,
             o_ref):
        s = s_ref[0] + s_ref[1]
        dg = d_ref[0, :, 0:1] + d_ref[1, :, 0:1]
        mean = s / jnp.maximum(dg, 1.0)
        hv = h_ref[...]
        h2 = jnp.dot(mean, wl_ref[...], preferred_element_type=jnp.float32)
        h2 = h2 + jnp.dot(hv, wr_ref[...], preferred_element_type=jnp.float32)
        h3 = jnp.maximum(hv + h2 + bl_ref[...], 0.0)
        o_ref[...] = jnp.dot(h3, wo_ref[...],
                             preferred_element_type=jnp.float32) + bo_ref[...]

    return pl.pallas_call(
        body,
        grid=(N // _R,),
        in_specs=[
            pl.BlockSpec((NC, _R, D), lambda i: (0, i, 0)),
            pl.BlockSpec((NC, _R, DEGW), lambda i: (0, i, 0)),
            pl.BlockSpec((_R, D), lambda i: (i, 0)),
            pl.BlockSpec((D, D), lambda i: (0, 0)),
            pl.BlockSpec((1, D), lambda i: (0, 0)),
            pl.BlockSpec((D, D), lambda i: (0, 0)),
            pl.BlockSpec((D, D), lambda i: (0, 0)),
            pl.BlockSpec((1, D), lambda i: (0, 0)),
        ],
        out_specs=pl.BlockSpec((_R, D), lambda i: (i, 0)),
        out_shape=jax.ShapeDtypeStruct((N, D), jnp.float32),
    )(sums, deg, h, W_l, b_l.reshape(1, D), W_r, W_lin, b_lin.reshape(1, D))


def kernel(x, edge_index, W_l1, b_l1, W_r1, W_l2, b_l2, W_r2, W_lin, b_lin):
    src = edge_index[0].astype(jnp.int32)
    dst = edge_index[1].astype(jnp.int32)
    npad = E_PAD - E
    src_p = jnp.concatenate([src, jnp.zeros((npad,), jnp.int32)])
    dst_p = jnp.concatenate([dst, jnp.full((npad,), N, jnp.int32)])
    zacc = jnp.zeros((NPAD, D), jnp.float32)
    zdeg = jnp.zeros((NPAD, DEGW), jnp.float32)
    ones_blk = jnp.ones((B, DEGW), jnp.float32)

    sums1, deg = _seg_sum_call(x, src_p, dst_p, zacc, zdeg, ones_blk, True)
    h = _tc_layer1(sums1, deg, x, W_l1, b_l1, W_r1)
    (sums2,) = _seg_sum_call(h, src_p, dst_p, zacc, None, None, False)
    out = _tc_layer2(sums2, deg, h, W_l2, b_l2, W_r2, W_lin, b_lin)
    return out


# speedup vs baseline: 3.6960x; 3.6960x over previous
import functools

import jax
import jax.numpy as jnp
from jax import lax
from jax.experimental import pallas as pl
from jax.experimental.pallas import tpu as pltpu
from jax.experimental.pallas import tpu_sc as plsc

N = 10000
E = 320000
D = 128

NC = 2
NS = 16
NW = NC * NS
B = 128
CHUNKS_PER_TILE = 79
E_PAD = NW * B * CHUNKS_PER_TILE
NPAD = 10112
ROWS_PER_SUB = NPAD // NS


def _seg_sum_call(feat, src_p, dst_p, zacc, ones_blk, with_deg):
    mesh = plsc.VectorSubcoreMesh(core_axis_name="c", subcore_axis_name="s")
    outs = [jax.ShapeDtypeStruct((NC, NPAD, D), jnp.float32)]
    scratch = [
        pltpu.VMEM((1, B), jnp.int32),
        pltpu.VMEM((1, B), jnp.int32),
        pltpu.VMEM((B, D), jnp.float32),
        pltpu.VMEM_SHARED((NPAD, D), jnp.float32),
        pltpu.SemaphoreType.DMA,
    ]
    if with_deg:
        outs.append(jax.ShapeDtypeStruct((NC, NPAD, D), jnp.float32))
        scratch.append(pltpu.VMEM((B, D), jnp.float32))

    def body(*refs):
        if with_deg:
            (feat_h, src_h, dst_h, zacc_h, ones_h,
             acc_out, deg_out,
             src_v, dst_v, rows_v, acc_sh, sem, ones_v) = refs
        else:
            (feat_h, src_h, dst_h, zacc_h,
             acc_out,
             src_v, dst_v, rows_v, acc_sh, sem) = refs
        cid = lax.axis_index("c")
        sid = lax.axis_index("s")
        wid = sid * NC + cid
        rz = ROWS_PER_SUB

        def zero_acc():
            pltpu.sync_copy(zacc_h.at[pl.ds(sid * rz, rz)],
                            acc_sh.at[pl.ds(sid * rz, rz)])

        zero_acc()
        if with_deg:
            pltpu.sync_copy(ones_h, ones_v)
        plsc.subcore_barrier()

        base = wid * CHUNKS_PER_TILE

        @pl.loop(0, CHUNKS_PER_TILE)
        def _(j):
            chunk = base + j
            pltpu.sync_copy(src_h.at[chunk], src_v)
            pltpu.sync_copy(dst_h.at[chunk], dst_v)
            pltpu.async_copy(feat_h.at[src_v.at[0]], rows_v, sem).wait()
            pltpu.sync_copy(rows_v, acc_sh.at[dst_v.at[0]], add=True)

        plsc.subcore_barrier()
        pltpu.sync_copy(acc_sh.at[pl.ds(sid * rz, rz)],
                        acc_out.at[cid, pl.ds(sid * rz, rz)])

        if with_deg:
            plsc.subcore_barrier()
            zero_acc()
            plsc.subcore_barrier()

            @pl.loop(0, CHUNKS_PER_TILE)
            def _(j):
                chunk = base + j
                pltpu.sync_copy(dst_h.at[chunk], dst_v)
                pltpu.sync_copy(ones_v, acc_sh.at[dst_v.at[0]], add=True)

            plsc.subcore_barrier()
            pltpu.sync_copy(acc_sh.at[pl.ds(sid * rz, rz)],
                            deg_out.at[cid, pl.ds(sid * rz, rz)])

    k = pl.kernel(body, out_type=tuple(outs), mesh=mesh,
                  scratch_types=tuple(scratch))
    if with_deg:
        return k(feat, src_p, dst_p, zacc, ones_blk)
    return k(feat, src_p, dst_p, zacc)


_R = 400


def _tc_layer1(sums, deg, x, W_l, b_l, W_r):
    def body(s_ref, d_ref, x_ref, wl_ref, bl_ref, wr_ref, o_ref):
        s = s_ref[0] + s_ref[1]
        dg = d_ref[0, :, 0:1] + d_ref[1, :, 0:1]
        mean = s / jnp.maximum(dg, 1.0)
        acc = jnp.dot(mean, wl_ref[...], preferred_element_type=jnp.float32)
        acc = acc + jnp.dot(x_ref[...], wr_ref[...],
                            preferred_element_type=jnp.float32)
        o_ref[...] = jnp.maximum(acc + bl_ref[...], 0.0)

    return pl.pallas_call(
        body,
        grid=(N // _R,),
        in_specs=[
            pl.BlockSpec((NC, _R, D), lambda i: (0, i, 0)),
            pl.BlockSpec((NC, _R, D), lambda i: (0, i, 0)),
            pl.BlockSpec((_R, D), lambda i: (i, 0)),
            pl.BlockSpec((D, D), lambda i: (0, 0)),
            pl.BlockSpec((1, D), lambda i: (0, 0)),
            pl.BlockSpec((D, D), lambda i: (0, 0)),
        ],
        out_specs=pl.BlockSpec((_R, D), lambda i: (i, 0)),
        out_shape=jax.ShapeDtypeStruct((N, D), jnp.float32),
    )(sums, deg, x, W_l, b_l.reshape(1, D), W_r)


def _tc_layer2(sums, deg, h, W_l, b_l, W_r, W_lin, b_lin):
    def body(s_ref, d_ref, h_ref, wl_ref, bl_ref, wr_ref, wo_ref, bo_ref,
             o_ref):
        s = s_ref[0] + s_ref[1]
        dg = d_ref[0, :, 0:1] + d_ref[1, :, 0:1]
        mean = s / jnp.maximum(dg, 1.0)
        hv = h_ref[...]
        h2 = jnp.dot(mean, wl_ref[...], preferred_element_type=jnp.float32)
        h2 = h2 + jnp.dot(hv, wr_ref[...], preferred_element_type=jnp.float32)
        h3 = jnp.maximum(hv + h2 + bl_ref[...], 0.0)
        o_ref[...] = jnp.dot(h3, wo_ref[...],
                             preferred_element_type=jnp.float32) + bo_ref[...]

    return pl.pallas_call(
        body,
        grid=(N // _R,),
        in_specs=[
            pl.BlockSpec((NC, _R, D), lambda i: (0, i, 0)),
            pl.BlockSpec((NC, _R, D), lambda i: (0, i, 0)),
            pl.BlockSpec((_R, D), lambda i: (i, 0)),
            pl.BlockSpec((D, D), lambda i: (0, 0)),
            pl.BlockSpec((1, D), lambda i: (0, 0)),
            pl.BlockSpec((D, D), lambda i: (0, 0)),
            pl.BlockSpec((D, D), lambda i: (0, 0)),
            pl.BlockSpec((1, D), lambda i: (0, 0)),
        ],
        out_specs=pl.BlockSpec((_R, D), lambda i: (i, 0)),
        out_shape=jax.ShapeDtypeStruct((N, D), jnp.float32),
    )(sums, deg, h, W_l, b_l.reshape(1, D), W_r, W_lin, b_lin.reshape(1, D))


def kernel(x, edge_index, W_l1, b_l1, W_r1, W_l2, b_l2, W_r2, W_lin, b_lin):
    src = edge_index[0].astype(jnp.int32)
    dst = edge_index[1].astype(jnp.int32)
    npad = E_PAD - E
    src_p = jnp.concatenate([src, jnp.zeros((npad,), jnp.int32)])
    src_p = src_p.reshape(E_PAD // B, 1, B)
    dst_p = jnp.concatenate([dst, jnp.full((npad,), N, jnp.int32)])
    dst_p = dst_p.reshape(E_PAD // B, 1, B)
    zacc = jnp.zeros((NPAD, D), jnp.float32)
    ones_blk = jnp.ones((B, D), jnp.float32)

    sums1, deg = _seg_sum_call(x, src_p, dst_p, zacc, ones_blk, True)
    h = _tc_layer1(sums1, deg, x, W_l1, b_l1, W_r1)
    (sums2,) = _seg_sum_call(h, src_p, dst_p, zacc, None, False)
    out = _tc_layer2(sums2, deg, h, W_l2, b_l2, W_r2, W_lin, b_lin)
    return out
